# baseline (speedup 1.0000x reference)
import jax
import jax.numpy as jnp
from jax.experimental import pallas as pl
from jax.experimental.pallas import tpu as pltpu

_TK = 2048


def _body(x_ref, g_ref, n_ref, vals_ref, idx_ref, m_ref):
    t = pl.program_id(0)
    x = x_ref[...]
    g = g_ref[...]
    s = 2.0 * jax.lax.dot_general(
        x, g, (((1,), (1,)), ((), ())),
        preferred_element_type=jnp.float32) - n_ref[0]
    local_max = jnp.max(s, axis=1, keepdims=True)
    lanes = jax.lax.broadcasted_iota(jnp.int32, s.shape, 1)
    big = jnp.int32(2**30)
    li = jnp.min(jnp.where(s == local_max, lanes, big),
                 axis=1, keepdims=True)
    oh = (lanes == li).astype(jnp.float32)
    local_vals = jax.lax.dot_general(
        oh, g, (((1,), (0,)), ((), ())),
        preferred_element_type=jnp.float32)
    local_arg = li + t * _TK

    @pl.when(t == 0)
    def _init():
        m_ref[...] = local_max
        idx_ref[...] = local_arg
        vals_ref[...] = local_vals

    @pl.when(t > 0)
    def _update():
        better = local_max > m_ref[...]
        m_ref[...] = jnp.where(better, local_max, m_ref[...])
        idx_ref[...] = jnp.where(better, local_arg, idx_ref[...])
        vals_ref[...] = jnp.where(better, local_vals, vals_ref[...])


def kernel(X, grid, grid_norm):
    B, D = X.shape
    N = grid.shape[0]
    T = N // _TK
    norm3 = grid_norm.reshape(T, 1, _TK)
    vals, idx = pl.pallas_call(
        _body,
        grid=(T,),
        in_specs=[
            pl.BlockSpec((B, D), lambda t: (0, 0)),
            pl.BlockSpec((_TK, D), lambda t: (t, 0)),
            pl.BlockSpec((1, 1, _TK), lambda t: (t, 0, 0)),
        ],
        out_specs=[
            pl.BlockSpec((B, D), lambda t: (0, 0)),
            pl.BlockSpec((B, 1), lambda t: (0, 0)),
        ],
        out_shape=[
            jax.ShapeDtypeStruct((B, D), jnp.float32),
            jax.ShapeDtypeStruct((B, 1), jnp.int32),
        ],
        scratch_shapes=[pltpu.VMEM((B, 1), jnp.float32)],
    )(X, grid, norm3)
    return vals, idx.reshape(B)


# speedup vs baseline: 1.6336x; 1.6336x over previous
import functools

import jax
import jax.numpy as jnp
from jax import lax
from jax.experimental import pallas as pl
from jax.experimental.pallas import tpu as pltpu
from jax.experimental.pallas import tpu_sc as plsc

_TK = 2048

_NC = 2
_NS = 16
_NW = _NC * _NS


def _scan_body(x_ref, g_ref, n_ref, idx_ref, m_ref):
    t = pl.program_id(0)
    x = x_ref[...]
    g = g_ref[...]
    s = 2.0 * jax.lax.dot_general(
        x, g, (((1,), (1,)), ((), ())),
        preferred_element_type=jnp.float32) - n_ref[0]
    local_max = jnp.max(s, axis=1, keepdims=True)
    lanes = jax.lax.broadcasted_iota(jnp.int32, s.shape, 1)
    big = jnp.int32(2**30)
    li = jnp.min(jnp.where(s == local_max, lanes, big),
                 axis=1, keepdims=True)
    local_arg = li + t * _TK

    @pl.when(t == 0)
    def _init():
        m_ref[...] = local_max
        idx_ref[...] = local_arg

    @pl.when(t > 0)
    def _update():
        better = local_max > m_ref[...]
        m_ref[...] = jnp.where(better, local_max, m_ref[...])
        idx_ref[...] = jnp.where(better, local_arg, idx_ref[...])


def _argmax_scan(X, grid, norm3):
    B, D = X.shape
    T = grid.shape[0] // _TK
    idx = pl.pallas_call(
        _scan_body,
        grid=(T,),
        in_specs=[
            pl.BlockSpec((B, D), lambda t: (0, 0)),
            pl.BlockSpec((_TK, D), lambda t: (t, 0)),
            pl.BlockSpec((1, 1, _TK), lambda t: (t, 0, 0)),
        ],
        out_specs=pl.BlockSpec((B, 1), lambda t: (0, 0)),
        out_shape=jax.ShapeDtypeStruct((B, 1), jnp.int32),
        scratch_shapes=[pltpu.VMEM((B, 1), jnp.float32)],
    )(X, grid, norm3)
    return idx.reshape(B)


def _make_sc_gather(B, D):
    rows_per_w = B // _NW
    cw_per_row = 128 // D
    mesh = plsc.VectorSubcoreMesh(core_axis_name="c", subcore_axis_name="s")

    @functools.partial(
        pl.kernel,
        mesh=mesh,
        compiler_params=pltpu.CompilerParams(needs_layout_passes=False),
        out_type=jax.ShapeDtypeStruct((B * D,), jnp.float32),
        scratch_types=[
            pltpu.VMEM((rows_per_w,), jnp.int32),
            pltpu.VMEM((rows_per_w,), jnp.int32),
            pltpu.VMEM((rows_per_w, 128), jnp.float32),
            pltpu.VMEM((rows_per_w * D,), jnp.float32),
            pltpu.SemaphoreType.DMA,
        ],
    )
    def gather(grid_hbm, idx_hbm, out_hbm, idx_v, ridx_v, rows_v, out_v, sem):
        wid = lax.axis_index("s") * _NC + lax.axis_index("c")
        base = wid * rows_per_w
        pltpu.sync_copy(idx_hbm.at[pl.ds(base, rows_per_w)], idx_v)
        cw_shift = cw_per_row.bit_length() - 1
        d_shift = D.bit_length() - 1
        for c in range(rows_per_w // 16):
            ridx_v[pl.ds(c * 16, 16)] = idx_v[pl.ds(c * 16, 16)] >> cw_shift
        pltpu.async_copy(grid_hbm.at[ridx_v], rows_v, sem).wait()

        def extract(j, carry):
            pos = j * 16 + lax.iota(jnp.int32, 16)
            rloc = pos >> d_shift
            ig = plsc.load_gather(idx_v, [rloc])
            col = ((ig & (cw_per_row - 1)) << d_shift) + (pos & (D - 1))
            out_v[pl.ds(j * 16, 16)] = plsc.load_gather(rows_v, [rloc, col])
            return carry

        lax.fori_loop(0, rows_per_w * D // 16, extract, 0)
        pltpu.sync_copy(out_v, out_hbm.at[pl.ds(base * D, rows_per_w * D)])

    return gather


def kernel(X, grid, grid_norm):
    B, D = X.shape
    T = grid.shape[0] // _TK
    norm3 = grid_norm.reshape(T, 1, _TK)
    idx = _argmax_scan(X, grid, norm3)
    vals = _make_sc_gather(B, D)(grid.reshape(-1, 128), idx)
    return vals.reshape(B, D), idx


# speedup vs baseline: 1.8320x; 1.1215x over previous
import functools

import jax
import jax.numpy as jnp
from jax import lax
from jax.experimental import pallas as pl
from jax.experimental.pallas import tpu as pltpu
from jax.experimental.pallas import tpu_sc as plsc

_TK = 2048

_NC = 2
_NS = 16
_NW = _NC * _NS


def _scan_body(x_ref, g_ref, idx_ref, m_ref):
    t = pl.program_id(0)
    s = jax.lax.dot_general(
        x_ref[...], g_ref[...], (((1,), (1,)), ((), ())),
        preferred_element_type=jnp.float32)
    local_max = jnp.max(s, axis=1, keepdims=True)
    lanes = jax.lax.broadcasted_iota(jnp.int32, s.shape, 1).astype(jnp.float32)
    big = jnp.float32(3e9)
    li = jnp.min(jnp.where(s == local_max, lanes, big),
                 axis=1, keepdims=True).astype(jnp.int32)
    local_arg = li + t * _TK

    @pl.when(t == 0)
    def _init():
        m_ref[...] = local_max
        idx_ref[...] = local_arg

    @pl.when(t > 0)
    def _update():
        better = local_max > m_ref[...]
        m_ref[...] = jnp.where(better, local_max, m_ref[...])
        idx_ref[...] = jnp.where(better, local_arg, idx_ref[...])


def _argmax_scan(Xa, Ga):
    B, K = Xa.shape
    T = Ga.shape[0] // _TK
    idx = pl.pallas_call(
        _scan_body,
        grid=(T,),
        in_specs=[
            pl.BlockSpec((B, K), lambda t: (0, 0)),
            pl.BlockSpec((_TK, K), lambda t: (t, 0)),
        ],
        out_specs=pl.BlockSpec((B, 1), lambda t: (0, 0)),
        out_shape=jax.ShapeDtypeStruct((B, 1), jnp.int32),
        scratch_shapes=[pltpu.VMEM((B, 1), jnp.float32)],
    )(Xa, Ga)
    return idx.reshape(B)


def _make_sc_gather(B, D):
    rows_per_w = B // _NW
    cw_per_row = 128 // D
    mesh = plsc.VectorSubcoreMesh(core_axis_name="c", subcore_axis_name="s")

    @functools.partial(
        pl.kernel,
        mesh=mesh,
        compiler_params=pltpu.CompilerParams(needs_layout_passes=False),
        out_type=jax.ShapeDtypeStruct((B * D,), jnp.float32),
        scratch_types=[
            pltpu.VMEM((rows_per_w,), jnp.int32),
            pltpu.VMEM((rows_per_w,), jnp.int32),
            pltpu.VMEM((rows_per_w, 128), jnp.float32),
            pltpu.VMEM((rows_per_w * D,), jnp.float32),
            pltpu.SemaphoreType.DMA,
        ],
    )
    def gather(grid_hbm, idx_hbm, out_hbm, idx_v, ridx_v, rows_v, out_v, sem):
        wid = lax.axis_index("s") * _NC + lax.axis_index("c")
        base = wid * rows_per_w
        pltpu.sync_copy(idx_hbm.at[pl.ds(base, rows_per_w)], idx_v)
        cw_shift = cw_per_row.bit_length() - 1
        d_shift = D.bit_length() - 1
        for c in range(rows_per_w // 16):
            ridx_v[pl.ds(c * 16, 16)] = idx_v[pl.ds(c * 16, 16)] >> cw_shift
        pltpu.async_copy(grid_hbm.at[ridx_v], rows_v, sem).wait()

        def extract(j, carry):
            pos = j * 16 + lax.iota(jnp.int32, 16)
            rloc = pos >> d_shift
            ig = plsc.load_gather(idx_v, [rloc])
            col = ((ig & (cw_per_row - 1)) << d_shift) + (pos & (D - 1))
            out_v[pl.ds(j * 16, 16)] = plsc.load_gather(rows_v, [rloc, col])
            return carry

        lax.fori_loop(0, rows_per_w * D // 16, extract, 0)
        pltpu.sync_copy(out_v, out_hbm.at[pl.ds(base * D, rows_per_w * D)])

    return gather


def kernel(X, grid, grid_norm):
    B, D = X.shape
    Xa = jnp.concatenate(
        [X + X, jnp.full((B, 1), -1.0, jnp.float32)], axis=1)
    Ga = jnp.concatenate([grid, grid_norm[:, None]], axis=1)
    idx = _argmax_scan(Xa, Ga)
    vals = _make_sc_gather(B, D)(grid.reshape(-1, 128), idx)
    return vals.reshape(B, D), idx


# speedup vs baseline: 1.8774x; 1.0248x over previous
import functools

import jax
import jax.numpy as jnp
from jax import lax
from jax.experimental import pallas as pl
from jax.experimental.pallas import tpu as pltpu
from jax.experimental.pallas import tpu_sc as plsc

_TK = 4096

_NC = 2
_NS = 16
_NW = _NC * _NS


def _scan_body(x_ref, g_ref, idx_ref, m_ref):
    t = pl.program_id(0)
    s = jax.lax.dot_general(
        x_ref[...], g_ref[...], (((1,), (1,)), ((), ())),
        preferred_element_type=jnp.float32)
    local_max = jnp.max(s, axis=1, keepdims=True)
    lanes = jax.lax.broadcasted_iota(jnp.int32, s.shape, 1).astype(jnp.float32)
    big = jnp.float32(3e9)
    li = jnp.min(jnp.where(s == local_max, lanes, big),
                 axis=1, keepdims=True).astype(jnp.int32)
    local_arg = li + t * _TK

    @pl.when(t == 0)
    def _init():
        m_ref[...] = local_max
        idx_ref[...] = local_arg

    @pl.when(t > 0)
    def _update():
        better = local_max > m_ref[...]
        m_ref[...] = jnp.where(better, local_max, m_ref[...])
        idx_ref[...] = jnp.where(better, local_arg, idx_ref[...])


def _argmax_scan(Xa, Ga):
    B, K = Xa.shape
    T = Ga.shape[0] // _TK
    idx = pl.pallas_call(
        _scan_body,
        grid=(T,),
        in_specs=[
            pl.BlockSpec((B, K), lambda t: (0, 0)),
            pl.BlockSpec((_TK, K), lambda t: (t, 0)),
        ],
        out_specs=pl.BlockSpec((B, 1), lambda t: (0, 0)),
        out_shape=jax.ShapeDtypeStruct((B, 1), jnp.int32),
        scratch_shapes=[pltpu.VMEM((B, 1), jnp.float32)],
    )(Xa, Ga)
    return idx.reshape(B)


def _make_sc_gather(B, D):
    rows_per_w = B // _NW
    cw_per_row = 128 // D
    mesh = plsc.VectorSubcoreMesh(core_axis_name="c", subcore_axis_name="s")

    @functools.partial(
        pl.kernel,
        mesh=mesh,
        compiler_params=pltpu.CompilerParams(needs_layout_passes=False),
        out_type=jax.ShapeDtypeStruct((B * D,), jnp.float32),
        scratch_types=[
            pltpu.VMEM((rows_per_w,), jnp.int32),
            pltpu.VMEM((rows_per_w,), jnp.int32),
            pltpu.VMEM((rows_per_w, 128), jnp.float32),
            pltpu.VMEM((rows_per_w * D,), jnp.float32),
            pltpu.SemaphoreType.DMA,
        ],
    )
    def gather(grid_hbm, idx_hbm, out_hbm, idx_v, ridx_v, rows_v, out_v, sem):
        wid = lax.axis_index("s") * _NC + lax.axis_index("c")
        base = wid * rows_per_w
        pltpu.sync_copy(idx_hbm.at[pl.ds(base, rows_per_w)], idx_v)
        cw_shift = cw_per_row.bit_length() - 1
        d_shift = D.bit_length() - 1
        for c in range(rows_per_w // 16):
            ridx_v[pl.ds(c * 16, 16)] = idx_v[pl.ds(c * 16, 16)] >> cw_shift
        pltpu.async_copy(grid_hbm.at[ridx_v], rows_v, sem).wait()

        def extract(j, carry):
            pos = j * 16 + lax.iota(jnp.int32, 16)
            rloc = pos >> d_shift
            ig = plsc.load_gather(idx_v, [rloc])
            col = ((ig & (cw_per_row - 1)) << d_shift) + (pos & (D - 1))
            out_v[pl.ds(j * 16, 16)] = plsc.load_gather(rows_v, [rloc, col])
            return carry

        lax.fori_loop(0, rows_per_w * D // 16, extract, 0)
        pltpu.sync_copy(out_v, out_hbm.at[pl.ds(base * D, rows_per_w * D)])

    return gather


def kernel(X, grid, grid_norm):
    B, D = X.shape
    Xa = jnp.concatenate(
        [X + X, jnp.full((B, 1), -1.0, jnp.float32)], axis=1)
    Ga = jnp.concatenate([grid, grid_norm[:, None]], axis=1)
    idx = _argmax_scan(Xa, Ga)
    vals = _make_sc_gather(B, D)(grid.reshape(-1, 128), idx)
    return vals.reshape(B, D), idx


# speedup vs baseline: 1.9364x; 1.0314x over previous
import functools

import jax
import jax.numpy as jnp
from jax import lax
from jax.experimental import pallas as pl
from jax.experimental.pallas import tpu as pltpu
from jax.experimental.pallas import tpu_sc as plsc

_TK = 4096

_NC = 2
_NS = 16
_NW = _NC * _NS


def _scan_body(x_ref, g_ref, idx_ref, m_ref):
    t = pl.program_id(0)
    x = x_ref[...]
    half = _TK // 2
    maxes = []
    args = []
    for h in range(2):
        g = g_ref[pl.ds(h * half, half), :]
        s = jax.lax.dot_general(
            x, g, (((1,), (1,)), ((), ())),
            preferred_element_type=jnp.float32)
        local_max = jnp.max(s, axis=1, keepdims=True)
        lanes = jax.lax.broadcasted_iota(
            jnp.int32, s.shape, 1).astype(jnp.float32)
        big = jnp.float32(3e9)
        li = jnp.min(jnp.where(s == local_max, lanes, big),
                     axis=1, keepdims=True).astype(jnp.int32)
        maxes.append(local_max)
        args.append(li + (t * _TK + h * half))
    hb = maxes[1] > maxes[0]
    local_max = jnp.where(hb, maxes[1], maxes[0])
    local_arg = jnp.where(hb, args[1], args[0])

    @pl.when(t == 0)
    def _init():
        m_ref[...] = local_max
        idx_ref[...] = local_arg

    @pl.when(t > 0)
    def _update():
        better = local_max > m_ref[...]
        m_ref[...] = jnp.where(better, local_max, m_ref[...])
        idx_ref[...] = jnp.where(better, local_arg, idx_ref[...])


def _argmax_scan(Xa, Ga):
    B, K = Xa.shape
    T = Ga.shape[0] // _TK
    idx = pl.pallas_call(
        _scan_body,
        grid=(T,),
        in_specs=[
            pl.BlockSpec((B, K), lambda t: (0, 0)),
            pl.BlockSpec((_TK, K), lambda t: (t, 0)),
        ],
        out_specs=pl.BlockSpec((B, 1), lambda t: (0, 0)),
        out_shape=jax.ShapeDtypeStruct((B, 1), jnp.int32),
        scratch_shapes=[pltpu.VMEM((B, 1), jnp.float32)],
    )(Xa, Ga)
    return idx.reshape(B)


def _make_sc_gather(B, D):
    rows_per_w = B // _NW
    cw_per_row = 128 // D
    mesh = plsc.VectorSubcoreMesh(core_axis_name="c", subcore_axis_name="s")

    @functools.partial(
        pl.kernel,
        mesh=mesh,
        compiler_params=pltpu.CompilerParams(needs_layout_passes=False),
        out_type=jax.ShapeDtypeStruct((B * D,), jnp.float32),
        scratch_types=[
            pltpu.VMEM((rows_per_w,), jnp.int32),
            pltpu.VMEM((rows_per_w,), jnp.int32),
            pltpu.VMEM((rows_per_w, 128), jnp.float32),
            pltpu.VMEM((rows_per_w * D,), jnp.float32),
            pltpu.SemaphoreType.DMA,
        ],
    )
    def gather(grid_hbm, idx_hbm, out_hbm, idx_v, ridx_v, rows_v, out_v, sem):
        wid = lax.axis_index("s") * _NC + lax.axis_index("c")
        base = wid * rows_per_w
        pltpu.sync_copy(idx_hbm.at[pl.ds(base, rows_per_w)], idx_v)
        cw_shift = cw_per_row.bit_length() - 1
        d_shift = D.bit_length() - 1
        for c in range(rows_per_w // 16):
            ridx_v[pl.ds(c * 16, 16)] = idx_v[pl.ds(c * 16, 16)] >> cw_shift
        pltpu.async_copy(grid_hbm.at[ridx_v], rows_v, sem).wait()

        def extract(j, carry):
            pos = j * 16 + lax.iota(jnp.int32, 16)
            rloc = pos >> d_shift
            ig = plsc.load_gather(idx_v, [rloc])
            col = ((ig & (cw_per_row - 1)) << d_shift) + (pos & (D - 1))
            out_v[pl.ds(j * 16, 16)] = plsc.load_gather(rows_v, [rloc, col])
            return carry

        lax.fori_loop(0, rows_per_w * D // 16, extract, 0)
        pltpu.sync_copy(out_v, out_hbm.at[pl.ds(base * D, rows_per_w * D)])

    return gather


def kernel(X, grid, grid_norm):
    B, D = X.shape
    Xa = jnp.concatenate(
        [X + X, jnp.full((B, 1), -1.0, jnp.float32)], axis=1)
    Ga = jnp.concatenate([grid, grid_norm[:, None]], axis=1)
    idx = _argmax_scan(Xa, Ga)
    vals = _make_sc_gather(B, D)(grid.reshape(-1, 128), idx)
    return vals.reshape(B, D), idx


# speedup vs baseline: 2.0361x; 1.0515x over previous
import functools

import jax
import jax.numpy as jnp
from jax import lax
from jax.experimental import pallas as pl
from jax.experimental.pallas import tpu as pltpu
from jax.experimental.pallas import tpu_sc as plsc

_TK = 4096

_NC = 2
_NS = 16
_NW = _NC * _NS


def _scan_body(x_ref, g_ref, idx_ref, m_ref):
    t = pl.program_id(0)
    x = x_ref[...]
    half = _TK // 4
    maxes = []
    args = []
    for h in range(4):
        g = g_ref[pl.ds(h * half, half), :]
        s = jax.lax.dot_general(
            x, g, (((1,), (1,)), ((), ())),
            preferred_element_type=jnp.float32)
        local_max = jnp.max(s, axis=1, keepdims=True)
        lanes = jax.lax.broadcasted_iota(
            jnp.int32, s.shape, 1).astype(jnp.float32)
        big = jnp.float32(3e9)
        li = jnp.min(jnp.where(s == local_max, lanes, big),
                     axis=1, keepdims=True).astype(jnp.int32)
        maxes.append(local_max)
        args.append(li + (t * _TK + h * half))
    local_max, local_arg = maxes[0], args[0]
    for h in range(1, 4):
        hb = maxes[h] > local_max
        local_max = jnp.where(hb, maxes[h], local_max)
        local_arg = jnp.where(hb, args[h], local_arg)

    @pl.when(t == 0)
    def _init():
        m_ref[...] = local_max
        idx_ref[...] = local_arg

    @pl.when(t > 0)
    def _update():
        better = local_max > m_ref[...]
        m_ref[...] = jnp.where(better, local_max, m_ref[...])
        idx_ref[...] = jnp.where(better, local_arg, idx_ref[...])


def _argmax_scan(Xa, Ga):
    B, K = Xa.shape
    T = Ga.shape[0] // _TK
    idx = pl.pallas_call(
        _scan_body,
        grid=(T,),
        in_specs=[
            pl.BlockSpec((B, K), lambda t: (0, 0)),
            pl.BlockSpec((_TK, K), lambda t: (t, 0)),
        ],
        out_specs=pl.BlockSpec((B, 1), lambda t: (0, 0)),
        out_shape=jax.ShapeDtypeStruct((B, 1), jnp.int32),
        scratch_shapes=[pltpu.VMEM((B, 1), jnp.float32)],
    )(Xa, Ga)
    return idx.reshape(B)


def _make_sc_gather(B, D):
    rows_per_w = B // _NW
    cw_per_row = 128 // D
    mesh = plsc.VectorSubcoreMesh(core_axis_name="c", subcore_axis_name="s")

    @functools.partial(
        pl.kernel,
        mesh=mesh,
        compiler_params=pltpu.CompilerParams(needs_layout_passes=False),
        out_type=jax.ShapeDtypeStruct((B * D,), jnp.float32),
        scratch_types=[
            pltpu.VMEM((rows_per_w,), jnp.int32),
            pltpu.VMEM((rows_per_w,), jnp.int32),
            pltpu.VMEM((rows_per_w, 128), jnp.float32),
            pltpu.VMEM((rows_per_w * D,), jnp.float32),
            pltpu.SemaphoreType.DMA,
        ],
    )
    def gather(grid_hbm, idx_hbm, out_hbm, idx_v, ridx_v, rows_v, out_v, sem):
        wid = lax.axis_index("s") * _NC + lax.axis_index("c")
        base = wid * rows_per_w
        pltpu.sync_copy(idx_hbm.at[pl.ds(base, rows_per_w)], idx_v)
        cw_shift = cw_per_row.bit_length() - 1
        d_shift = D.bit_length() - 1
        for c in range(rows_per_w // 16):
            ridx_v[pl.ds(c * 16, 16)] = idx_v[pl.ds(c * 16, 16)] >> cw_shift
        pltpu.async_copy(grid_hbm.at[ridx_v], rows_v, sem).wait()

        def extract(j, carry):
            pos = j * 16 + lax.iota(jnp.int32, 16)
            rloc = pos >> d_shift
            ig = plsc.load_gather(idx_v, [rloc])
            col = ((ig & (cw_per_row - 1)) << d_shift) + (pos & (D - 1))
            out_v[pl.ds(j * 16, 16)] = plsc.load_gather(rows_v, [rloc, col])
            return carry

        lax.fori_loop(0, rows_per_w * D // 16, extract, 0)
        pltpu.sync_copy(out_v, out_hbm.at[pl.ds(base * D, rows_per_w * D)])

    return gather


def kernel(X, grid, grid_norm):
    B, D = X.shape
    Xa = jnp.concatenate(
        [X + X, jnp.full((B, 1), -1.0, jnp.float32)], axis=1)
    Ga = jnp.concatenate([grid, grid_norm[:, None]], axis=1)
    idx = _argmax_scan(Xa, Ga)
    vals = _make_sc_gather(B, D)(grid.reshape(-1, 128), idx)
    return vals.reshape(B, D), idx
